# baseline (speedup 1.0000x reference)
import functools

import jax
import jax.numpy as jnp
from jax import lax
from jax.experimental import pallas as pl
from jax.experimental.pallas import tpu as pltpu
from jax.experimental.pallas import tpu_sc as plsc

HIDDEN = 1024
ROUTES = 8
BITS = 4
NGRAM = 4
ALPHA = 2 ** BITS
EMBED = 128
ROWS = ROUTES * ALPHA ** NGRAM

NUM_CORES = 2
NUM_SUBCORES = 16
NW = NUM_CORES * NUM_SUBCORES
LANES = 16

CHUNK_POS = 16


def _route_body(x_ref, wr_ref, conf_ref, idx_ref):
    T = x_ref.shape[0]
    logits = jnp.dot(x_ref[...], wr_ref[...],
                     preferred_element_type=jnp.float32)
    cb = 1.0 / (1.0 + jnp.exp(-jnp.abs(logits)))
    logcb = jnp.log(cb)
    bits = (logits > 0.0).astype(jnp.float32)
    row = lax.broadcasted_iota(jnp.int32, (ROUTES * BITS, ROUTES), 0)
    col = lax.broadcasted_iota(jnp.int32, (ROUTES * BITS, ROUTES), 1)
    sel = (row // BITS == col).astype(jnp.float32)
    gw = sel * (2.0 ** (row % BITS).astype(jnp.float32))
    conf = jnp.exp(jnp.dot(logcb, sel, preferred_element_type=jnp.float32))
    codes = jnp.dot(bits, gw, preferred_element_type=jnp.float32)
    addr = codes
    zrow = jnp.zeros((1, ROUTES), jnp.float32)
    shifted = codes
    for k in range(1, NGRAM):
        shifted = jnp.concatenate([zrow, shifted[:T - 1]], axis=0)
        addr = addr + shifted * float(ALPHA ** k)
    route_off = lax.broadcasted_iota(jnp.int32, (T, ROUTES), 1) * (ALPHA ** NGRAM)
    conf_ref[...] = conf
    idx_ref[...] = addr.astype(jnp.int32) + route_off


def _routing(x2, w_route, batch, seq):
    return pl.pallas_call(
        _route_body,
        grid=(batch,),
        in_specs=[
            pl.BlockSpec((seq, HIDDEN), lambda b: (b, 0)),
            pl.BlockSpec((HIDDEN, ROUTES * BITS), lambda b: (0, 0)),
        ],
        out_specs=[
            pl.BlockSpec((seq, ROUTES), lambda b: (b, 0)),
            pl.BlockSpec((seq, ROUTES), lambda b: (b, 0)),
        ],
        out_shape=[
            jax.ShapeDtypeStruct((batch * seq, ROUTES), jnp.float32),
            jax.ShapeDtypeStruct((batch * seq, ROUTES), jnp.int32),
        ],
    )(x2, w_route)


_GDN = lax.GatherDimensionNumbers(
    offset_dims=(), collapsed_slice_dims=(0,), start_index_map=(0,))


def _lane_broadcast(v, lane):
    idx = jnp.full((LANES, 1), lane, jnp.int32)
    return lax.gather(v, idx, dimension_numbers=_GDN, slice_sizes=(1,),
                      mode=lax.GatherScatterMode.PROMISE_IN_BOUNDS)


def _make_pool_kernel(num_pos):
    pos_w = num_pos // NW
    nch = pos_w // CHUNK_POS
    idx_per_chunk = CHUNK_POS * ROUTES

    mesh = plsc.VectorSubcoreMesh(
        core_axis_name="c", subcore_axis_name="s",
        num_cores=NUM_CORES, num_subcores=NUM_SUBCORES)

    nbuf = 4

    @functools.partial(
        pl.kernel, mesh=mesh,
        out_type=jax.ShapeDtypeStruct((num_pos, EMBED), jnp.float32),
        scratch_types=[
            pltpu.VMEM((nch, idx_per_chunk), jnp.int32),
            pltpu.VMEM((nch, idx_per_chunk), jnp.float32),
            [pltpu.VMEM((idx_per_chunk, EMBED), jnp.float32)] * nbuf,
            [pltpu.VMEM((CHUNK_POS, EMBED), jnp.float32)] * 2,
            [pltpu.SemaphoreType.DMA] * nbuf,
            [pltpu.SemaphoreType.DMA] * 2,
        ],
    )
    def pool_kernel(idx_hbm, conf_hbm, table_hbm, out_hbm,
                    idx_v, conf_v, rows, pools, gsems, ssems):
        wid = lax.axis_index("s") * NUM_CORES + lax.axis_index("c")
        pltpu.sync_copy(idx_hbm.at[pl.ds(wid * nch, nch)], idx_v)
        pltpu.sync_copy(conf_hbm.at[pl.ds(wid * nch, nch)], conf_v)

        def _out_slice(c):
            return out_hbm.at[pl.ds(wid * pos_w + c * CHUNK_POS, CHUNK_POS)]

        for b in range(nbuf - 1):
            pltpu.async_copy(table_hbm.at[idx_v.at[b]], rows[b], gsems[b])

        def _compute(c, rows_v, pool_v):
            @pl.loop(0, CHUNK_POS // 2)
            def _pair_pos(pp):
                cv = conf_v[c, pl.ds(pp * LANES, LANES)]
                for q in range(2):
                    p = pp * 2 + q
                    accs = [None] * (EMBED // LANES)
                    for r in range(ROUTES):
                        cs = _lane_broadcast(cv, q * ROUTES + r)
                        rrow = p * ROUTES + r
                        for j in range(EMBED // LANES):
                            v = cs * rows_v[rrow, pl.ds(j * LANES, LANES)]
                            accs[j] = v if accs[j] is None else accs[j] + v
                    for j in range(EMBED // LANES):
                        pool_v[p, pl.ds(j * LANES, LANES)] = accs[j]

        @pl.loop(0, nch, step=nbuf)
        def _group(g):
            for b in range(nbuf):
                c = g + b
                pb = b % 2

                @pl.when(c + nbuf - 1 < nch)
                def _():
                    pltpu.async_copy(table_hbm.at[idx_v.at[c + nbuf - 1]],
                                     rows[(b + nbuf - 1) % nbuf],
                                     gsems[(b + nbuf - 1) % nbuf])

                pltpu.make_async_copy(table_hbm.at[idx_v.at[c]],
                                      rows[b], gsems[b]).wait()

                @pl.when(c >= 2)
                def _():
                    pltpu.make_async_copy(pools[pb], _out_slice(c),
                                          ssems[pb]).wait()

                _compute(c, rows[b], pools[pb])
                pltpu.async_copy(pools[pb], _out_slice(c), ssems[pb])

        for pb in range(2):
            pltpu.make_async_copy(pools[pb], _out_slice(nch - 2 + pb),
                                  ssems[pb]).wait()

    return pool_kernel


def _proj_body(p_ref, w_ref, o_ref):
    o_ref[...] = jnp.dot(p_ref[...], w_ref[...],
                         preferred_element_type=jnp.float32)


def _project(pooled, w_out, num_pos):
    blk = 4096
    return pl.pallas_call(
        _proj_body,
        grid=(num_pos // blk,),
        in_specs=[
            pl.BlockSpec((blk, EMBED), lambda i: (i, 0)),
            pl.BlockSpec((EMBED, HIDDEN), lambda i: (0, 0)),
        ],
        out_specs=pl.BlockSpec((blk, HIDDEN), lambda i: (i, 0)),
        out_shape=jax.ShapeDtypeStruct((num_pos, HIDDEN), jnp.float32),
    )(pooled, w_out)


def kernel(x, W_route, table, W_out):
    B, T, D = x.shape
    num_pos = B * T
    x2 = x.reshape(num_pos, D)
    conf, idx = _routing(x2, W_route, B, T)
    nrow = num_pos * ROUTES // 128
    pooled = _make_pool_kernel(num_pos)(idx.reshape(nrow, 128),
                                        conf.reshape(nrow, 128), table)
    out = _project(pooled, W_out, num_pos)
    return out.reshape(B, T, HIDDEN)


# speedup vs baseline: 1.1356x; 1.1356x over previous
import functools

import jax
import jax.numpy as jnp
from jax import lax
from jax.experimental import pallas as pl
from jax.experimental.pallas import tpu as pltpu
from jax.experimental.pallas import tpu_sc as plsc

HIDDEN = 1024
ROUTES = 8
BITS = 4
NGRAM = 4
ALPHA = 2 ** BITS
EMBED = 128
ROWS = ROUTES * ALPHA ** NGRAM

NUM_CORES = 2
NUM_SUBCORES = 16
NW = NUM_CORES * NUM_SUBCORES
LANES = 16

TILE_POS = 128
QPOS = 32


def _route_body(x_ref, wr_ref, conf_ref, idx_ref):
    T = x_ref.shape[0]
    logits = jnp.dot(x_ref[...], wr_ref[...],
                     preferred_element_type=jnp.float32)
    cb = 1.0 / (1.0 + jnp.exp(-jnp.abs(logits)))
    logcb = jnp.log(cb)
    bits = (logits > 0.0).astype(jnp.float32)
    row = lax.broadcasted_iota(jnp.int32, (ROUTES * BITS, ROUTES), 0)
    col = lax.broadcasted_iota(jnp.int32, (ROUTES * BITS, ROUTES), 1)
    sel = (row // BITS == col).astype(jnp.float32)
    gw = sel * (2.0 ** (row % BITS).astype(jnp.float32))
    logconf = jnp.dot(logcb, sel, preferred_element_type=jnp.float32)
    codes = jnp.dot(bits, gw, preferred_element_type=jnp.float32)
    conf_t = jnp.exp(jnp.transpose(logconf, (1, 0)))
    codes_t = jnp.transpose(codes, (1, 0))
    addr = codes_t
    zcol = jnp.zeros((ROUTES, 1), jnp.float32)
    shifted = codes_t
    for k in range(1, NGRAM):
        shifted = jnp.concatenate([zcol, shifted[:, :T - 1]], axis=1)
        addr = addr + shifted * float(ALPHA ** k)
    route_off = lax.broadcasted_iota(jnp.int32, (ROUTES, T), 0) * (ALPHA ** NGRAM)
    conf_ref[...] = conf_t
    idx_ref[...] = addr.astype(jnp.int32) + route_off


def _routing(x2, w_route, batch, seq):
    return pl.pallas_call(
        _route_body,
        grid=(batch,),
        in_specs=[
            pl.BlockSpec((seq, HIDDEN), lambda b: (b, 0)),
            pl.BlockSpec((HIDDEN, ROUTES * BITS), lambda b: (0, 0)),
        ],
        out_specs=[
            pl.BlockSpec((ROUTES, seq), lambda b: (0, b)),
            pl.BlockSpec((ROUTES, seq), lambda b: (0, b)),
        ],
        out_shape=[
            jax.ShapeDtypeStruct((ROUTES, batch * seq), jnp.float32),
            jax.ShapeDtypeStruct((ROUTES, batch * seq), jnp.int32),
        ],
    )(x2, w_route)


_GDN = lax.GatherDimensionNumbers(
    offset_dims=(), collapsed_slice_dims=(0,), start_index_map=(0,))


def _lane_broadcast(v, lane):
    idx = jnp.full((LANES, 1), lane, jnp.int32)
    return lax.gather(v, idx, dimension_numbers=_GDN, slice_sizes=(1,),
                      mode=lax.GatherScatterMode.PROMISE_IN_BOUNDS)


def _make_pool_kernel(num_pos):
    pos_w = num_pos // NW
    nrow_w = pos_w * ROUTES // TILE_POS
    nq = pos_w // QPOS
    qper = TILE_POS // QPOS

    mesh = plsc.VectorSubcoreMesh(
        core_axis_name="c", subcore_axis_name="s",
        num_cores=NUM_CORES, num_subcores=NUM_SUBCORES)

    @functools.partial(
        pl.kernel, mesh=mesh,
        out_type=jax.ShapeDtypeStruct((num_pos, EMBED), jnp.float32),
        scratch_types=[
            pltpu.VMEM((nrow_w, TILE_POS), jnp.int32),
            pltpu.VMEM((nrow_w, TILE_POS), jnp.float32),
            [[pltpu.VMEM((QPOS, EMBED), jnp.float32)] * ROUTES] * 2,
            [pltpu.VMEM((QPOS, EMBED), jnp.float32)] * 2,
            [pltpu.SemaphoreType.DMA] * 2,
            [pltpu.SemaphoreType.DMA] * 2,
        ],
    )
    def pool_kernel(idx_hbm, conf_hbm, table_hbm, out_hbm,
                    idx_v, conf_v, rows, pools, gsems, ssems):
        wid = lax.axis_index("s") * NUM_CORES + lax.axis_index("c")
        pltpu.sync_copy(idx_hbm.at[pl.ds(wid * nrow_w, nrow_w)], idx_v)
        pltpu.sync_copy(conf_hbm.at[pl.ds(wid * nrow_w, nrow_w)], conf_v)

        def _gather_quarter(qt, par):
            ct = lax.div(qt, qper)
            off = lax.rem(qt, qper) * QPOS
            for r in range(ROUTES):
                pltpu.async_copy(
                    table_hbm.at[idx_v.at[ct * ROUTES + r, pl.ds(off, QPOS)]],
                    rows[par][r], gsems[par])

        def _drain_quarter(qt, par):
            ct = lax.div(qt, qper)
            off = lax.rem(qt, qper) * QPOS
            for r in range(ROUTES):
                pltpu.make_async_copy(
                    table_hbm.at[idx_v.at[ct * ROUTES + r, pl.ds(off, QPOS)]],
                    rows[par][r], gsems[par]).wait()

        def _out_slice(qt):
            return out_hbm.at[pl.ds(wid * pos_w + qt * QPOS, QPOS)]

        _gather_quarter(0, 0)

        @pl.loop(0, nq, step=2)
        def _qpair(g):
            for par in range(2):
                qt = g + par
                ct = lax.div(qt, qper)
                off = lax.rem(qt, qper) * QPOS

                @pl.when(qt + 1 < nq)
                def _():
                    _gather_quarter(qt + 1, 1 - par)

                _drain_quarter(qt, par)

                @pl.when(qt >= 2)
                def _():
                    pltpu.make_async_copy(pools[par], _out_slice(qt),
                                          ssems[par]).wait()

                @pl.loop(0, QPOS // LANES)
                def _pgroup(pp):
                    cvs = [conf_v[ct * ROUTES + r,
                                  pl.ds(off + pp * LANES, LANES)]
                           for r in range(ROUTES)]

                    @pl.loop(0, LANES)
                    def _pos(q):
                        p = pp * LANES + q
                        accs = [None] * (EMBED // LANES)
                        for r in range(ROUTES):
                            cs = _lane_broadcast(cvs[r], q)
                            for j in range(EMBED // LANES):
                                v = cs * rows[par][r][p, pl.ds(j * LANES,
                                                               LANES)]
                                accs[j] = (v if accs[j] is None
                                           else accs[j] + v)
                        for j in range(EMBED // LANES):
                            pools[par][p, pl.ds(j * LANES, LANES)] = accs[j]

                pltpu.async_copy(pools[par], _out_slice(qt), ssems[par])

        for par in range(2):
            pltpu.make_async_copy(pools[par], _out_slice(nq - 2 + par),
                                  ssems[par]).wait()

    return pool_kernel


def _proj_body(p_ref, w_ref, o_ref):
    o_ref[...] = jnp.dot(p_ref[...], w_ref[...],
                         preferred_element_type=jnp.float32)


def _project(pooled, w_out, num_pos):
    blk = 2048
    return pl.pallas_call(
        _proj_body,
        grid=(num_pos // blk,),
        in_specs=[
            pl.BlockSpec((blk, EMBED), lambda i: (i, 0)),
            pl.BlockSpec((EMBED, HIDDEN), lambda i: (0, 0)),
        ],
        out_specs=pl.BlockSpec((blk, HIDDEN), lambda i: (i, 0)),
        out_shape=jax.ShapeDtypeStruct((num_pos, HIDDEN), jnp.float32),
    )(pooled, w_out)


def kernel(x, W_route, table, W_out):
    B, T, D = x.shape
    num_pos = B * T
    x2 = x.reshape(num_pos, D)
    conf, idx = _routing(x2, W_route, B, T)
    nt = num_pos // 128
    def _rows_view(a):
        return a.reshape(ROUTES, nt, 128).transpose(1, 0, 2).reshape(
            nt * ROUTES, 128)
    pooled = _make_pool_kernel(num_pos)(_rows_view(idx), _rows_view(conf),
                                        table)
    out = _project(pooled, W_out, num_pos)
    return out.reshape(B, T, HIDDEN)
